# baseline (speedup 1.0000x reference)
import functools

import jax
from jax import lax
import jax.numpy as jnp
from jax.experimental import pallas as pl
from jax.experimental.pallas import tpu as pltpu
from jax.experimental.pallas import tpu_sc as plsc

_MAX_REL = 32
_NC = 2
_NS = 16
_L = 16


def _bias_sc_body(w_hbm, bias_hbm, w_v, p_v, out_v, *, row0, seq, d, rmax,
                  rows_per_w):
    nk = 2 * rmax + 1
    wid = lax.axis_index("s") * _NC + lax.axis_index("c")
    base = row0 + wid * rows_per_w
    pltpu.sync_copy(w_hbm, w_v)
    nchunk = d // _L
    for c in range(nchunk):
        p_v[0, pl.ds(c * _L, _L)] = jnp.zeros((_L,), jnp.float32)

    def step(k, carry):
        for c in range(nchunk):
            sl = pl.ds(c * _L, _L)
            p_v[k + 1, sl] = p_v[k, sl] + w_v[k, sl]
        return carry

    lax.fori_loop(0, nk, step, 0)
    inv = 1.0 / seq
    for r in range(rows_per_w):
        i = base + r
        a = jnp.maximum(i - (rmax - 1), 0).astype(jnp.float32) * inv
        b = jnp.maximum(seq - rmax - i, 0).astype(jnp.float32) * inv
        kmin = jnp.maximum(1, rmax - i)
        khi = jnp.minimum(2 * rmax - 1, seq + rmax - 1 - i) + 1
        for c in range(nchunk):
            sl = pl.ds(c * _L, _L)
            row = (
                a * w_v[0, sl]
                + b * w_v[2 * rmax, sl]
                + (p_v[khi, sl] - p_v[kmin, sl]) * inv
            )
            out_v[r, sl] = row
    pltpu.sync_copy(out_v, bias_hbm.at[pl.ds(wid * rows_per_w, rows_per_w)])


def _compute_bias_sc(w, row0, nrows, seq, d):
    rows_per_w = nrows // (_NC * _NS)
    nk = 2 * _MAX_REL + 1
    return pl.kernel(
        functools.partial(
            _bias_sc_body, row0=row0, seq=seq, d=d, rmax=_MAX_REL,
            rows_per_w=rows_per_w,
        ),
        out_type=jax.ShapeDtypeStruct((nrows, d), jnp.float32),
        mesh=plsc.VectorSubcoreMesh(core_axis_name="c", subcore_axis_name="s"),
        scratch_types=[
            pltpu.VMEM((nk, d), jnp.float32),
            pltpu.VMEM((nk + 1, d), jnp.float32),
            pltpu.VMEM((rows_per_w, d), jnp.float32),
        ],
    )(w)


def _add_body_a(x_ref, w_ref, o_ref, *, bt, seq, rmax):
    i = pl.program_id(0)
    nk = 2 * rmax + 1
    b = jax.lax.broadcasted_iota(jnp.int32, (bt, nk), 0) + i * bt
    k = jax.lax.broadcasted_iota(jnp.int32, (bt, nk), 1)
    j = b + (k - rmax)
    interior = ((k > 0) & (k < 2 * rmax) & (j >= 0) & (j < seq)).astype(jnp.int32)
    counts = jnp.where(
        k == 0,
        jnp.maximum(b - (rmax - 1), 0),
        jnp.where(k == 2 * rmax, jnp.maximum(seq - rmax - b, 0), interior),
    ).astype(jnp.float32)
    bias = jnp.dot(counts, w_ref[...], preferred_element_type=jnp.float32)
    bias = bias * (1.0 / seq)
    o_ref[...] = x_ref[...] + bias[:, None, :]


def _add_body_b(x_ref, b_ref, a_ref, o_ref):
    del a_ref
    o_ref[...] = x_ref[...] + b_ref[...][:, None, :]


def kernel(x, rel_pos_emb_weight):
    batch, seq, d = x.shape
    half = 5 * batch // 32
    bias = _compute_bias_sc(rel_pos_emb_weight, 0, batch, seq, d)
    bt = 16
    na = half // bt
    out_shape = jax.ShapeDtypeStruct((batch, seq, d), x.dtype)
    a_out = pl.pallas_call(
        functools.partial(_add_body_a, bt=bt, seq=seq, rmax=_MAX_REL),
        grid=(na,),
        in_specs=[
            pl.BlockSpec((bt, seq, d), lambda i: (i, 0, 0)),
            pl.BlockSpec(rel_pos_emb_weight.shape, lambda i: (0, 0)),
        ],
        out_specs=pl.BlockSpec((bt, seq, d), lambda i: (i, 0, 0)),
        out_shape=out_shape,
        compiler_params=pltpu.CompilerParams(
            dimension_semantics=("arbitrary",),
        ),
    )(x, rel_pos_emb_weight)
    nb = (batch - half) // bt
    return pl.pallas_call(
        _add_body_b,
        grid=(nb,),
        in_specs=[
            pl.BlockSpec((bt, seq, d), lambda i: (i + na, 0, 0)),
            pl.BlockSpec((bt, d), lambda i: (i + na, 0)),
            pl.BlockSpec(memory_space=pltpu.MemorySpace.HBM),
        ],
        out_specs=pl.BlockSpec((bt, seq, d), lambda i: (i + na, 0, 0)),
        out_shape=out_shape,
        input_output_aliases={2: 0},
        compiler_params=pltpu.CompilerParams(
            dimension_semantics=("arbitrary",),
        ),
    )(x, bias, a_out)


# speedup vs baseline: 1.0052x; 1.0052x over previous
import functools

import jax
from jax import lax
import jax.numpy as jnp
from jax.experimental import pallas as pl
from jax.experimental.pallas import tpu as pltpu
from jax.experimental.pallas import tpu_sc as plsc

_MAX_REL = 32
_NC = 2
_NS = 16
_L = 16


def _bias_sc_body(w_hbm, bias_hbm, w_v, p_v, out_v, *, row0, seq, d, rmax,
                  rows_per_w):
    nk = 2 * rmax + 1
    wid = lax.axis_index("s") * _NC + lax.axis_index("c")
    base = row0 + wid * rows_per_w
    pltpu.sync_copy(w_hbm, w_v)
    nchunk = d // _L
    for c in range(nchunk):
        p_v[0, pl.ds(c * _L, _L)] = jnp.zeros((_L,), jnp.float32)

    def step(k, carry):
        for c in range(nchunk):
            sl = pl.ds(c * _L, _L)
            p_v[k + 1, sl] = p_v[k, sl] + w_v[k, sl]
        return carry

    lax.fori_loop(0, nk, step, 0)
    inv = 1.0 / seq
    for r in range(rows_per_w):
        i = base + r
        a = jnp.maximum(i - (rmax - 1), 0).astype(jnp.float32) * inv
        b = jnp.maximum(seq - rmax - i, 0).astype(jnp.float32) * inv
        kmin = jnp.maximum(1, rmax - i)
        khi = jnp.minimum(2 * rmax - 1, seq + rmax - 1 - i) + 1
        for c in range(nchunk):
            sl = pl.ds(c * _L, _L)
            row = (
                a * w_v[0, sl]
                + b * w_v[2 * rmax, sl]
                + (p_v[khi, sl] - p_v[kmin, sl]) * inv
            )
            out_v[r, sl] = row
    pltpu.sync_copy(out_v, bias_hbm.at[pl.ds(wid * rows_per_w, rows_per_w)])


def _compute_bias_sc(w, row0, nrows, seq, d):
    rows_per_w = nrows // (_NC * _NS)
    nk = 2 * _MAX_REL + 1
    return pl.kernel(
        functools.partial(
            _bias_sc_body, row0=row0, seq=seq, d=d, rmax=_MAX_REL,
            rows_per_w=rows_per_w,
        ),
        out_type=jax.ShapeDtypeStruct((nrows, d), jnp.float32),
        mesh=plsc.VectorSubcoreMesh(core_axis_name="c", subcore_axis_name="s"),
        scratch_types=[
            pltpu.VMEM((nk, d), jnp.float32),
            pltpu.VMEM((nk + 1, d), jnp.float32),
            pltpu.VMEM((rows_per_w, d), jnp.float32),
        ],
    )(w)


def _add_body_a(x_ref, w_ref, o_ref, *, bt, seq, rmax):
    i = pl.program_id(0)
    nk = 2 * rmax + 1
    b = jax.lax.broadcasted_iota(jnp.int32, (bt, nk), 0) + i * bt
    k = jax.lax.broadcasted_iota(jnp.int32, (bt, nk), 1)
    j = b + (k - rmax)
    interior = ((k > 0) & (k < 2 * rmax) & (j >= 0) & (j < seq)).astype(jnp.int32)
    counts = jnp.where(
        k == 0,
        jnp.maximum(b - (rmax - 1), 0),
        jnp.where(k == 2 * rmax, jnp.maximum(seq - rmax - b, 0), interior),
    ).astype(jnp.float32)
    bias = jnp.dot(counts, w_ref[...], preferred_element_type=jnp.float32)
    bias = bias * (1.0 / seq)
    o_ref[...] = x_ref[...] + bias[:, None, :]


def _add_body_b(x_ref, b_ref, a_ref, o_ref):
    del a_ref
    o_ref[...] = x_ref[...] + b_ref[...][:, None, :]


def kernel(x, rel_pos_emb_weight):
    batch, seq, d = x.shape
    half = 3 * batch // 16
    bias = _compute_bias_sc(rel_pos_emb_weight, 0, batch, seq, d)
    bt = 16
    na = half // bt
    out_shape = jax.ShapeDtypeStruct((batch, seq, d), x.dtype)
    a_out = pl.pallas_call(
        functools.partial(_add_body_a, bt=bt, seq=seq, rmax=_MAX_REL),
        grid=(na,),
        in_specs=[
            pl.BlockSpec((bt, seq, d), lambda i: (i, 0, 0)),
            pl.BlockSpec(rel_pos_emb_weight.shape, lambda i: (0, 0)),
        ],
        out_specs=pl.BlockSpec((bt, seq, d), lambda i: (i, 0, 0)),
        out_shape=out_shape,
        compiler_params=pltpu.CompilerParams(
            dimension_semantics=("parallel",),
        ),
    )(x, rel_pos_emb_weight)
    nb = (batch - half) // bt
    return pl.pallas_call(
        _add_body_b,
        grid=(nb,),
        in_specs=[
            pl.BlockSpec((bt, seq, d), lambda i: (i + na, 0, 0)),
            pl.BlockSpec((bt, d), lambda i: (i + na, 0)),
            pl.BlockSpec(memory_space=pltpu.MemorySpace.HBM),
        ],
        out_specs=pl.BlockSpec((bt, seq, d), lambda i: (i + na, 0, 0)),
        out_shape=out_shape,
        input_output_aliases={2: 0},
        compiler_params=pltpu.CompilerParams(
            dimension_semantics=("parallel",),
        ),
    )(x, bias, a_out)


# speedup vs baseline: 1.0076x; 1.0024x over previous
import functools

import jax
from jax import lax
import jax.numpy as jnp
from jax.experimental import pallas as pl
from jax.experimental.pallas import tpu as pltpu
from jax.experimental.pallas import tpu_sc as plsc

_MAX_REL = 32
_NC = 2
_NS = 16
_L = 16


def _bias_sc_body(w_hbm, bias_hbm, w_v, p_v, out_v, *, row0, seq, d, rmax,
                  rows_per_w):
    nk = 2 * rmax + 1
    wid = lax.axis_index("s") * _NC + lax.axis_index("c")
    base = row0 + wid * rows_per_w
    pltpu.sync_copy(w_hbm, w_v)
    nchunk = d // _L
    for c in range(nchunk):
        p_v[0, pl.ds(c * _L, _L)] = jnp.zeros((_L,), jnp.float32)

    def step(k, carry):
        for c in range(nchunk):
            sl = pl.ds(c * _L, _L)
            p_v[k + 1, sl] = p_v[k, sl] + w_v[k, sl]
        return carry

    lax.fori_loop(0, nk, step, 0)
    inv = 1.0 / seq
    for r in range(rows_per_w):
        i = base + r
        a = jnp.maximum(i - (rmax - 1), 0).astype(jnp.float32) * inv
        b = jnp.maximum(seq - rmax - i, 0).astype(jnp.float32) * inv
        kmin = jnp.maximum(1, rmax - i)
        khi = jnp.minimum(2 * rmax - 1, seq + rmax - 1 - i) + 1
        for c in range(nchunk):
            sl = pl.ds(c * _L, _L)
            row = (
                a * w_v[0, sl]
                + b * w_v[2 * rmax, sl]
                + (p_v[khi, sl] - p_v[kmin, sl]) * inv
            )
            out_v[r, sl] = row
    pltpu.sync_copy(out_v, bias_hbm.at[pl.ds(wid * rows_per_w, rows_per_w)])


def _compute_bias_sc(w, row0, nrows, seq, d):
    rows_per_w = nrows // (_NC * _NS)
    nk = 2 * _MAX_REL + 1
    return pl.kernel(
        functools.partial(
            _bias_sc_body, row0=row0, seq=seq, d=d, rmax=_MAX_REL,
            rows_per_w=rows_per_w,
        ),
        out_type=jax.ShapeDtypeStruct((nrows, d), jnp.float32),
        mesh=plsc.VectorSubcoreMesh(core_axis_name="c", subcore_axis_name="s"),
        scratch_types=[
            pltpu.VMEM((nk, d), jnp.float32),
            pltpu.VMEM((nk + 1, d), jnp.float32),
            pltpu.VMEM((rows_per_w, d), jnp.float32),
        ],
    )(w)


def _add_body_a(x_ref, w_ref, o_ref, *, bt, seq, rmax):
    i = pl.program_id(0)
    nk = 2 * rmax + 1
    b = jax.lax.broadcasted_iota(jnp.int32, (bt, nk), 0) + i * bt
    k = jax.lax.broadcasted_iota(jnp.int32, (bt, nk), 1)
    j = b + (k - rmax)
    interior = ((k > 0) & (k < 2 * rmax) & (j >= 0) & (j < seq)).astype(jnp.int32)
    counts = jnp.where(
        k == 0,
        jnp.maximum(b - (rmax - 1), 0),
        jnp.where(k == 2 * rmax, jnp.maximum(seq - rmax - b, 0), interior),
    ).astype(jnp.float32)
    bias = jnp.dot(counts, w_ref[...], preferred_element_type=jnp.float32)
    bias = bias * (1.0 / seq)
    o_ref[...] = x_ref[...] + bias[:, None, :]


def _add_body_b(x_ref, b_ref, a_ref, o_ref):
    del a_ref
    o_ref[...] = x_ref[...] + b_ref[...][:, None, :]


def kernel(x, rel_pos_emb_weight):
    batch, seq, d = x.shape
    half = 3 * batch // 16
    bias = _compute_bias_sc(rel_pos_emb_weight, 0, batch, seq, d)
    bt = 16
    na = half // bt
    out_shape = jax.ShapeDtypeStruct((batch, seq, d), x.dtype)
    a_out = pl.pallas_call(
        functools.partial(_add_body_a, bt=bt, seq=seq, rmax=_MAX_REL),
        grid=(na,),
        in_specs=[
            pl.BlockSpec((bt, seq, d), lambda i: (i, 0, 0)),
            pl.BlockSpec(rel_pos_emb_weight.shape, lambda i: (0, 0)),
        ],
        out_specs=pl.BlockSpec((bt, seq, d), lambda i: (i, 0, 0)),
        out_shape=out_shape,
        compiler_params=pltpu.CompilerParams(
            dimension_semantics=("arbitrary",),
        ),
    )(x, rel_pos_emb_weight)
    nb = (batch - half) // bt
    return pl.pallas_call(
        _add_body_b,
        grid=(nb,),
        in_specs=[
            pl.BlockSpec((bt, seq, d), lambda i: (i + na, 0, 0)),
            pl.BlockSpec((bt, d), lambda i: (i + na, 0)),
            pl.BlockSpec(memory_space=pltpu.MemorySpace.HBM),
        ],
        out_specs=pl.BlockSpec((bt, seq, d), lambda i: (i + na, 0, 0)),
        out_shape=out_shape,
        input_output_aliases={2: 0},
        compiler_params=pltpu.CompilerParams(
            dimension_semantics=("arbitrary",),
        ),
    )(x, bias, a_out)
